# baseline (speedup 1.0000x reference)
import functools

import jax
import jax.numpy as jnp
from jax import lax
from jax.experimental import pallas as pl
from jax.experimental.pallas import tpu as pltpu
from jax.experimental.pallas import tpu_sc as plsc

_NC, _NS = 2, 16
_NW = _NC * _NS
_CHUNK = 128
_NSLOT = 4


def _c_body(tok_ref, pos_ref, c_ref):
    pos = pos_ref[...]
    V = tok_ref.shape[0]
    S = pos.shape[0]
    for v in range(V):
        c_ref[pl.ds(v * S, S), :] = pos + tok_ref[v][None]


def _build_c(token_emb, pos_emb):
    V, D = token_emb.shape
    S = pos_emb.shape[0]
    return pl.pallas_call(
        _c_body,
        out_shape=jax.ShapeDtypeStruct((V * S, D), jnp.float32),
    )(token_emb, pos_emb)


def _sc_body(rows_per_w, n_chunks, x_hbm, c_hbm, out_hbm, x_v, idx_v,
             r0, r1, r2, r3, g0, g1, g2, g3, w0, w1, w2, w3):
    rows = [r0, r1, r2, r3]
    gsems = [g0, g1, g2, g3]
    wsems = [w0, w1, w2, w3]

    wid = lax.axis_index("s") * _NC + lax.axis_index("c")
    base = wid * rows_per_w
    pltpu.sync_copy(x_hbm.at[pl.ds(base, rows_per_w)], x_v)

    iota = lax.iota(jnp.int32, 16)

    def idx_body(c, carry):
        for k in range(_CHUNK // 16):
            v = x_v[pl.ds(c * _CHUNK + k * 16, 16)]
            idx_v[c, pl.ds(k * 16, 16)] = v * 128 + (iota + k * 16)
        return carry

    lax.fori_loop(0, n_chunks, idx_body, 0)

    def start_g(i, c):
        pltpu.async_copy(c_hbm.at[idx_v.at[c]], rows[i], gsems[i])

    def wait_g(i, c):
        pltpu.make_async_copy(c_hbm.at[idx_v.at[c]], rows[i], gsems[i]).wait()

    def start_w(i, c):
        pltpu.async_copy(rows[i], out_hbm.at[pl.ds(base + c * _CHUNK, _CHUNK)],
                         wsems[i])

    def wait_w(i, c):
        pltpu.make_async_copy(rows[i],
                              out_hbm.at[pl.ds(base + c * _CHUNK, _CHUNK)],
                              wsems[i]).wait()

    def round_body(r, carry):
        for i in range(_NSLOT):
            c = r * _NSLOT + i

            @pl.when(r > 0)
            def _drain():
                wait_w(i, c)

            start_g(i, c)
        for i in range(_NSLOT):
            c = r * _NSLOT + i
            wait_g(i, c)
            start_w(i, c)
        return carry

    lax.fori_loop(0, n_chunks // _NSLOT, round_body, 0)
    for i in range(_NSLOT):
        wait_w(i, 0)


def kernel(x, token_emb, pos_emb):
    x = x.astype(jnp.int32)
    B, S = x.shape
    V, D = token_emb.shape
    c_tab = _build_c(token_emb, pos_emb)

    n_rows = B * S
    rows_per_w = n_rows // _NW
    n_chunks = rows_per_w // _CHUNK
    xf = x.reshape(n_rows)

    mesh = plsc.VectorSubcoreMesh(core_axis_name="c", subcore_axis_name="s",
                                  num_cores=_NC, num_subcores=_NS)
    body = functools.partial(_sc_body, rows_per_w, n_chunks)
    out = pl.kernel(
        body,
        out_type=jax.ShapeDtypeStruct((n_rows, D), jnp.float32),
        mesh=mesh,
        scratch_types=[
            pltpu.VMEM((rows_per_w,), jnp.int32),
            pltpu.VMEM((n_chunks, _CHUNK), jnp.int32),
        ] + [pltpu.VMEM((_CHUNK, D), jnp.float32)] * _NSLOT
          + [pltpu.SemaphoreType.DMA] * (2 * _NSLOT),
    )(xf, c_tab)
    return out.reshape(B, S, D)
